# baseline (speedup 1.0000x reference)
import functools

import jax
import jax.numpy as jnp
from jax import lax
from jax.experimental import pallas as pl
from jax.experimental.pallas import tpu as pltpu
from jax.experimental.pallas import tpu_sc as plsc

_BSL = 256
_NSL = 2
_NT = 50
_D = 64


_S0 = 499968
_PROWS = 1000000 - _S0


def _tc_pack(ta_ref, tb_ref, out_ref):
    out_ref[...] = jnp.concatenate([ta_ref[...].T, tb_ref[...].T], axis=1)


def _compute_indices(idx_all, t, sl, idsh, par, iota):
    for m in range(_BSL // 16):
        v = idx_all[t, pl.ds(_BSL * sl + 16 * m, 16)]
        hi = v >= _S0
        idsh[pl.ds(16 * m, 16)] = jnp.where(hi, v - _S0, v)
        par[pl.ds(16 * m, 16)] = jnp.where(hi, _D, 0)


def _transpose_to_tiles(g_ref, par_ref, tr_ref, iota):
    for jj in range(_BSL // 128):
        for cs in range(8):
            bl_vec = iota + (128 * jj + 16 * cs)
            par_vec = par_ref[pl.ds(128 * jj + 16 * cs, 16)]

            @plsc.parallel_loop(0, _D)
            def body(h):
                vals = plsc.load_gather(g_ref, [bl_vec, par_vec + h])
                tr_ref[h, pl.ds(128 * jj + 16 * cs, 16)] = vals


def _emb_kernel(num_cores, idx_hbm, table_hbm, out_hbm,
                idx_all, g_a, g_b, tr, ish_a, ish_b, par_a, par_b,
                sg_a, sg_b, ss):
    wid = lax.axis_index("s") * num_cores + lax.axis_index("c")
    b0 = wid * (_NSL * _BSL)
    iota = lax.iota(jnp.int32, 16)
    nstep = _NT * _NSL

    pltpu.sync_copy(idx_hbm.at[:, pl.ds(b0, _NSL * _BSL)], idx_all)

    def prep_and_gather(s, ish, par, g, sem):
        s = jnp.minimum(s, nstep - 1)
        t = s % _NT
        sl = s // _NT
        _compute_indices(idx_all, t, sl, ish, par, iota)
        pltpu.async_copy(table_hbm.at[ish], g, sem)

    def wait_gather(g, sem):
        pltpu.make_async_copy(table_hbm.at[ish_a], g, sem).wait()

    def out_slice(s):
        t = s % _NT
        sl = s // _NT
        return out_hbm.at[pl.ds(_D * t, _D), pl.ds(b0 + _BSL * sl, _BSL)]

    def wait_store():
        pltpu.make_async_copy(tr, out_slice(0), ss).wait()

    def step(s, g, sem, ish, par, first):
        wait_gather(g, sem)
        if not first:
            wait_store()
        _transpose_to_tiles(g, par, tr, iota)
        pltpu.async_copy(tr, out_slice(s), ss)
        prep_and_gather(s + 2, ish, par, g, sem)

    prep_and_gather(0, ish_a, par_a, g_a, sg_a)
    prep_and_gather(1, ish_b, par_b, g_b, sg_b)
    step(0, g_a, sg_a, ish_a, par_a, True)
    step(1, g_b, sg_b, ish_b, par_b, False)

    def body(k, carry):
        step(2 + 2 * k, g_a, sg_a, ish_a, par_a, False)
        step(3 + 2 * k, g_b, sg_b, ish_b, par_b, False)
        return carry

    lax.fori_loop(0, (nstep - 2) // 2, body, 0)
    wait_store()
    wait_gather(g_a, sg_a)
    wait_gather(g_b, sg_b)


def kernel(input_ids, table):
    B, S = input_ids.shape
    V, D = table.shape
    idx2d = input_ids.T.astype(jnp.int32)
    tt = table.T

    tablep = pl.pallas_call(
        _tc_pack,
        grid=((_PROWS + 127) // 128,),
        in_specs=[pl.BlockSpec((_D, 128), lambda i: (0, i)),
                  pl.BlockSpec((_D, 128), lambda i: (0, i + _S0 // 128))],
        out_specs=pl.BlockSpec((128, 2 * _D), lambda i: (i, 0)),
        out_shape=jax.ShapeDtypeStruct((_PROWS, 2 * D), jnp.float32),
    )(tt, tt)

    info = plsc.get_sparse_core_info()
    nw = info.num_cores * info.num_subcores
    assert B == nw * _NSL * _BSL and S == _NT and D == _D

    mesh = plsc.VectorSubcoreMesh(core_axis_name="c", subcore_axis_name="s")
    emb = functools.partial(
        pl.kernel,
        mesh=mesh,
        out_type=jax.ShapeDtypeStruct((S * D, B), jnp.float32),
        scratch_types=[
            pltpu.VMEM((_NT, _NSL * _BSL), jnp.int32),
            pltpu.VMEM((_BSL, 2 * _D), jnp.float32),
            pltpu.VMEM((_BSL, 2 * _D), jnp.float32),
            pltpu.VMEM((_D, _BSL), jnp.float32),
            pltpu.VMEM((_BSL,), jnp.int32),
            pltpu.VMEM((_BSL,), jnp.int32),
            pltpu.VMEM((_BSL,), jnp.int32),
            pltpu.VMEM((_BSL,), jnp.int32),
            pltpu.SemaphoreType.DMA,
            pltpu.SemaphoreType.DMA,
            pltpu.SemaphoreType.DMA,
        ],
        compiler_params=pltpu.CompilerParams(use_tc_tiling_on_sc=True,
                                             needs_layout_passes=False),
    )(functools.partial(_emb_kernel, info.num_cores))

    o = emb(idx2d, tablep)
    return o.reshape(S, D, B).transpose(2, 0, 1)


# speedup vs baseline: 2.5958x; 2.5958x over previous
import functools

import jax
import jax.numpy as jnp
from jax import lax
from jax.experimental import pallas as pl
from jax.experimental.pallas import tpu as pltpu
from jax.experimental.pallas import tpu_sc as plsc

_BSL = 512
_NT = 50
_D = 64


def _transpose_to_tiles(g_flat, tr_flat, flat_vecs, zero16):
    @plsc.parallel_loop(0, _D, unroll=2)
    def body(ir):
        i = ir >> 3
        r = ir & 7
        col_vec = zero16 + ir
        for j in range(4):
            for cs in range(8):
                vals = plsc.load_gather(g_flat, [flat_vecs[8 * j + cs],
                                                 col_vec])
                tr_flat[i, j, r, pl.ds(16 * cs, 16)] = vals


def _emb_kernel(num_cores, idx_hbm, table_hbm, out_hbm,
                idx_all, g_a, g_b, tr, sg_a, sg_b, ss):
    wid = lax.axis_index("s") * num_cores + lax.axis_index("c")
    b0 = wid * _BSL
    cblk = wid * (_BSL // 128)
    iota = lax.iota(jnp.int32, 16)
    zero16 = jnp.zeros((16,), jnp.int32)
    flat_vecs = [iota + (128 * j + 16 * cs)
                 for j in range(4) for cs in range(8)]

    pltpu.sync_copy(idx_hbm.at[:, pl.ds(b0, _BSL)], idx_all)

    def start_gather(t, g, sem):
        pltpu.async_copy(table_hbm.at[idx_all.at[t]], g, sem)

    def wait_gather(g, sem):
        pltpu.make_async_copy(table_hbm.at[idx_all.at[0]], g, sem).wait()

    def wait_store():
        pltpu.make_async_copy(
            tr, out_hbm.at[pl.ds(0, 8), pl.ds(cblk, 4)], ss).wait()

    def step(t, g, sem, first):
        wait_gather(g, sem)
        if not first:
            wait_store()
        _transpose_to_tiles(g, tr, flat_vecs, zero16)
        pltpu.async_copy(tr, out_hbm.at[pl.ds(8 * t, 8), pl.ds(cblk, 4)], ss)
        start_gather(jnp.minimum(t + 2, _NT - 1), g, sem)

    start_gather(0, g_a, sg_a)
    start_gather(1, g_b, sg_b)
    step(0, g_a, sg_a, True)
    step(1, g_b, sg_b, False)

    def body(k, carry):
        step(2 + 2 * k, g_a, sg_a, False)
        step(3 + 2 * k, g_b, sg_b, False)
        return carry

    lax.fori_loop(0, (_NT - 2) // 2, body, 0)
    wait_store()
    wait_gather(g_a, sg_a)
    wait_gather(g_b, sg_b)


def kernel(input_ids, table):
    B, S = input_ids.shape
    V, D = table.shape
    idx2d = input_ids.T.astype(jnp.int32)

    info = plsc.get_sparse_core_info()
    nw = info.num_cores * info.num_subcores
    assert B == nw * _BSL and S == _NT and D == _D

    mesh = plsc.VectorSubcoreMesh(core_axis_name="c", subcore_axis_name="s")
    emb = functools.partial(
        pl.kernel,
        mesh=mesh,
        out_type=jax.ShapeDtypeStruct((S * 8, B // 128, 8, 128), jnp.float32),
        scratch_types=[
            pltpu.VMEM((_NT, _BSL), jnp.int32),
            pltpu.VMEM((_BSL, _D), jnp.float32),
            pltpu.VMEM((_BSL, _D), jnp.float32),
            pltpu.VMEM((8, _BSL // 128, 8, 128), jnp.float32),
            pltpu.SemaphoreType.DMA,
            pltpu.SemaphoreType.DMA,
            pltpu.SemaphoreType.DMA,
        ],
        compiler_params=pltpu.CompilerParams(use_tc_tiling_on_sc=False,
                                             needs_layout_passes=False),
    )(functools.partial(_emb_kernel, info.num_cores))

    i1 = emb(idx2d, table)
    out = (i1.reshape(S, 8, B // 128, 8, 128)
             .transpose(2, 4, 0, 1, 3)
             .reshape(B, S, D))
    return out


# speedup vs baseline: 2.7204x; 1.0480x over previous
import functools

import jax
import jax.numpy as jnp
from jax import lax
from jax.experimental import pallas as pl
from jax.experimental.pallas import tpu as pltpu
from jax.experimental.pallas import tpu_sc as plsc

CHUNK = 512


def _emb_kernel(n_per_w, n_chunks, num_cores, idx_hbm, table_hbm, out_hbm,
                idx_v, r0, r1, sg0, sg1, ss0, ss1):
    wid = lax.axis_index("s") * num_cores + lax.axis_index("c")
    base = wid * n_per_w

    pltpu.sync_copy(idx_hbm.at[pl.ds(base, n_per_w)], idx_v)

    def start_gather(c, r, sem):
        pltpu.async_copy(table_hbm.at[idx_v.at[pl.ds(c * CHUNK, CHUNK)]],
                         r, sem)

    def wait_gather(r, sem):
        pltpu.make_async_copy(
            table_hbm.at[idx_v.at[pl.ds(0, CHUNK)]], r, sem).wait()

    def start_store(c, r, sem):
        pltpu.async_copy(r, out_hbm.at[pl.ds(base + c * CHUNK, CHUNK)], sem)

    def wait_store(r, sem):
        pltpu.make_async_copy(r, out_hbm.at[pl.ds(base, CHUNK)], sem).wait()

    last = n_chunks - 1
    start_gather(0, r0, sg0)
    start_gather(1, r1, sg1)

    def body(g, carry):
        a = 2 * g
        wait_gather(r0, sg0)
        start_store(a, r0, ss0)
        wait_gather(r1, sg1)
        start_store(a + 1, r1, ss1)
        c0 = jnp.minimum(a + 2, last)
        c1 = jnp.minimum(a + 3, last)
        wait_store(r0, ss0)
        start_gather(c0, r0, sg0)
        wait_store(r1, ss1)
        start_gather(c1, r1, sg1)
        return carry

    lax.fori_loop(0, n_chunks // 2, body, 0)
    wait_gather(r0, sg0)
    wait_gather(r1, sg1)


def kernel(input_ids, table):
    B, S = input_ids.shape
    V, D = table.shape
    N = B * S
    idx = input_ids.reshape(N).astype(jnp.int32)

    info = plsc.get_sparse_core_info()
    nw = info.num_cores * info.num_subcores
    n_per_w = N // nw
    assert n_per_w * nw == N and n_per_w % (2 * CHUNK) == 0
    n_chunks = n_per_w // CHUNK

    mesh = plsc.VectorSubcoreMesh(core_axis_name="c", subcore_axis_name="s")
    emb = functools.partial(
        pl.kernel,
        mesh=mesh,
        out_type=jax.ShapeDtypeStruct((N, D), jnp.float32),
        scratch_types=[
            pltpu.VMEM((n_per_w,), jnp.int32),
            pltpu.VMEM((CHUNK, D), jnp.float32),
            pltpu.VMEM((CHUNK, D), jnp.float32),
            pltpu.SemaphoreType.DMA,
            pltpu.SemaphoreType.DMA,
            pltpu.SemaphoreType.DMA,
            pltpu.SemaphoreType.DMA,
        ],
        compiler_params=pltpu.CompilerParams(use_tc_tiling_on_sc=False),
    )(functools.partial(_emb_kernel, n_per_w, n_chunks, info.num_cores))

    out = emb(idx, table)
    return out.reshape(B, S, D)
